# baseline (speedup 1.0000x reference)
import functools

import jax
import jax.numpy as jnp
from jax.experimental import pallas as pl


def _encode_kernel(xm_ref, w_ref, b_ref, out_ref, lo_ref, hi_ref):
    f = pl.program_id(0)
    acc = jnp.dot(xm_ref[...], w_ref[...], preferred_element_type=jnp.float32)
    post = jnp.maximum(acc + b_ref[...], 0.0)
    out_ref[...] = post
    tmax = jnp.broadcast_to(jnp.max(post, axis=1, keepdims=True), lo_ref.shape)

    @pl.when(f == 0)
    def _():
        lo_ref[...] = tmax
        hi_ref[...] = tmax

    @pl.when(f != 0)
    def _():
        lo_ref[...] = jnp.minimum(lo_ref[...], tmax)
        hi_ref[...] = jnp.maximum(hi_ref[...], tmax)


def _threshold_kernel(post_ref, lo_ref, hi_ref, tau_ref, *, k, use_lo):
    v = jax.lax.bitcast_convert_type(post_ref[...], jnp.int32)
    if use_lo:
        lo = jax.lax.bitcast_convert_type(lo_ref[:, :1], jnp.int32)
    else:
        lo = jnp.zeros((v.shape[0], 1), jnp.int32)
    hi = jax.lax.bitcast_convert_type(hi_ref[:, :1], jnp.int32) + 1

    def cond(carry):
        lo, hi = carry
        return jnp.any(hi - lo > 1)

    def body(carry):
        lo, hi = carry
        mid = lo + (hi - lo) // 2
        cnt = jnp.sum((v >= mid).astype(jnp.int32), axis=1, keepdims=True)
        ge = cnt >= k
        hi = jnp.where(cnt == k, mid + 1, jnp.where(ge, hi, mid))
        lo = jnp.where(ge, mid, lo)
        return (lo, hi)

    lo, hi = jax.lax.while_loop(cond, body, (lo, hi))
    tau_ref[...] = jax.lax.bitcast_convert_type(lo, jnp.float32)


def _decode_kernel(post_ref, tau_ref, w_ref, bdec_ref, out_ref):
    f = pl.program_id(0)

    @pl.when(f == 0)
    def _():
        out_ref[...] = jnp.broadcast_to(bdec_ref[...], out_ref.shape)

    p = post_ref[...]
    masked = jnp.where(p >= tau_ref[...], p, 0.0)
    out_ref[...] += jnp.dot(masked, w_ref[...], preferred_element_type=jnp.float32)


@jax.jit
def kernel(x, W_enc, W_dec, encoder_bias, b_dec):
    ntok, act_dim = x.shape
    dict_size = W_enc.shape[1]
    k = 64

    f_tile = 1024
    fd_tile = 512
    n_ftiles = dict_size // f_tile
    xm = x - b_dec[None, :]
    bias2d = encoder_bias[None, :]

    post_relu, row_lo, row_hi = pl.pallas_call(
        _encode_kernel,
        grid=(n_ftiles,),
        in_specs=[
            pl.BlockSpec((ntok, act_dim), lambda f: (0, 0)),
            pl.BlockSpec((act_dim, f_tile), lambda f: (0, f)),
            pl.BlockSpec((1, f_tile), lambda f: (0, f)),
        ],
        out_specs=[
            pl.BlockSpec((ntok, f_tile), lambda f: (0, f)),
            pl.BlockSpec((ntok, 128), lambda f: (0, 0)),
            pl.BlockSpec((ntok, 128), lambda f: (0, 0)),
        ],
        out_shape=[
            jax.ShapeDtypeStruct((ntok, dict_size), jnp.float32),
            jax.ShapeDtypeStruct((ntok, 128), jnp.float32),
            jax.ShapeDtypeStruct((ntok, 128), jnp.float32),
        ],
    )(xm, W_enc, bias2d)

    t_tile = 128
    tau = pl.pallas_call(
        functools.partial(_threshold_kernel, k=k, use_lo=n_ftiles >= k),
        grid=(ntok // t_tile,),
        in_specs=[
            pl.BlockSpec((t_tile, dict_size), lambda t: (t, 0)),
            pl.BlockSpec((t_tile, 128), lambda t: (t, 0)),
            pl.BlockSpec((t_tile, 128), lambda t: (t, 0)),
        ],
        out_specs=pl.BlockSpec((t_tile, 1), lambda t: (t, 0)),
        out_shape=jax.ShapeDtypeStruct((ntok, 1), jnp.float32),
    )(post_relu, row_lo, row_hi)

    x_hat = pl.pallas_call(
        _decode_kernel,
        grid=(dict_size // fd_tile,),
        in_specs=[
            pl.BlockSpec((ntok, fd_tile), lambda f: (0, f)),
            pl.BlockSpec((ntok, 1), lambda f: (0, 0)),
            pl.BlockSpec((fd_tile, act_dim), lambda f: (f, 0)),
            pl.BlockSpec((1, act_dim), lambda f: (0, 0)),
        ],
        out_specs=pl.BlockSpec((ntok, act_dim), lambda f: (0, 0)),
        out_shape=jax.ShapeDtypeStruct((ntok, act_dim), jnp.float32),
    )(post_relu, tau, W_dec, b_dec[None, :])

    return x_hat


# speedup vs baseline: 1.1886x; 1.1886x over previous
import functools

import jax
import jax.numpy as jnp
from jax.experimental import pallas as pl


def _encode_kernel(xm_ref, w_ref, b_ref, out_ref, lo_ref, hi_ref):
    f = pl.program_id(0)
    acc = jnp.dot(xm_ref[...], w_ref[...], preferred_element_type=jnp.float32)
    post = jnp.maximum(acc + b_ref[...], 0.0)
    out_ref[...] = post
    tmax = jnp.broadcast_to(jnp.max(post, axis=1, keepdims=True), lo_ref.shape)

    @pl.when(f == 0)
    def _():
        lo_ref[...] = tmax
        hi_ref[...] = tmax

    @pl.when(f != 0)
    def _():
        lo_ref[...] = jnp.minimum(lo_ref[...], tmax)
        hi_ref[...] = jnp.maximum(hi_ref[...], tmax)


def _threshold_kernel(post_ref, lo_ref, hi_ref, tau_ref, *, k, use_lo):
    v = jax.lax.bitcast_convert_type(post_ref[...], jnp.int32)
    if use_lo:
        lo = jax.lax.bitcast_convert_type(lo_ref[:, :1], jnp.int32)
    else:
        lo = jnp.zeros((v.shape[0], 1), jnp.int32)
    hi = jax.lax.bitcast_convert_type(hi_ref[:, :1], jnp.int32) + 1

    def cond(carry):
        lo, hi = carry
        return jnp.any(hi - lo > 1)

    def body(carry):
        lo, hi = carry
        mid = lo + (hi - lo) // 2
        cnt = jnp.sum((v >= mid).astype(jnp.int32), axis=1, keepdims=True)
        ge = cnt >= k
        hi = jnp.where(cnt == k, mid + 1, jnp.where(ge, hi, mid))
        lo = jnp.where(ge, mid, lo)
        return (lo, hi)

    lo, hi = jax.lax.while_loop(cond, body, (lo, hi))
    tau_ref[...] = jax.lax.bitcast_convert_type(lo, jnp.float32)


def _decode_kernel(post_ref, tau_ref, w_ref, bdec_ref, out_ref):
    f = pl.program_id(0)

    @pl.when(f == 0)
    def _():
        out_ref[...] = jnp.broadcast_to(bdec_ref[...], out_ref.shape)

    p = post_ref[...]
    masked = jnp.where(p >= tau_ref[...], p, 0.0)
    out_ref[...] += jnp.dot(masked, w_ref[...], preferred_element_type=jnp.float32)


@jax.jit
def kernel(x, W_enc, W_dec, encoder_bias, b_dec):
    ntok, act_dim = x.shape
    dict_size = W_enc.shape[1]
    k = 64

    f_tile = 512
    fd_tile = 512
    n_ftiles = dict_size // f_tile
    xm = x - b_dec[None, :]
    bias2d = encoder_bias[None, :]

    post_relu, row_lo, row_hi = pl.pallas_call(
        _encode_kernel,
        grid=(n_ftiles,),
        in_specs=[
            pl.BlockSpec((ntok, act_dim), lambda f: (0, 0)),
            pl.BlockSpec((act_dim, f_tile), lambda f: (0, f)),
            pl.BlockSpec((1, f_tile), lambda f: (0, f)),
        ],
        out_specs=[
            pl.BlockSpec((ntok, f_tile), lambda f: (0, f)),
            pl.BlockSpec((ntok, 128), lambda f: (0, 0)),
            pl.BlockSpec((ntok, 128), lambda f: (0, 0)),
        ],
        out_shape=[
            jax.ShapeDtypeStruct((ntok, dict_size), jnp.float32),
            jax.ShapeDtypeStruct((ntok, 128), jnp.float32),
            jax.ShapeDtypeStruct((ntok, 128), jnp.float32),
        ],
    )(xm, W_enc, bias2d)

    t_tile = 128
    tau = pl.pallas_call(
        functools.partial(_threshold_kernel, k=k, use_lo=n_ftiles >= k),
        grid=(ntok // t_tile,),
        in_specs=[
            pl.BlockSpec((t_tile, dict_size), lambda t: (t, 0)),
            pl.BlockSpec((t_tile, 128), lambda t: (t, 0)),
            pl.BlockSpec((t_tile, 128), lambda t: (t, 0)),
        ],
        out_specs=pl.BlockSpec((t_tile, 1), lambda t: (t, 0)),
        out_shape=jax.ShapeDtypeStruct((ntok, 1), jnp.float32),
    )(post_relu, row_lo, row_hi)

    x_hat = pl.pallas_call(
        _decode_kernel,
        grid=(dict_size // fd_tile,),
        in_specs=[
            pl.BlockSpec((ntok, fd_tile), lambda f: (0, f)),
            pl.BlockSpec((ntok, 1), lambda f: (0, 0)),
            pl.BlockSpec((fd_tile, act_dim), lambda f: (f, 0)),
            pl.BlockSpec((1, act_dim), lambda f: (0, 0)),
        ],
        out_specs=pl.BlockSpec((ntok, act_dim), lambda f: (0, 0)),
        out_shape=jax.ShapeDtypeStruct((ntok, act_dim), jnp.float32),
    )(post_relu, tau, W_dec, b_dec[None, :])

    return x_hat


# speedup vs baseline: 1.2068x; 1.0153x over previous
import functools

import jax
import jax.numpy as jnp
from jax.experimental import pallas as pl


def _encode_kernel(xm_ref, w_ref, b_ref, out_ref, lo_ref, hi_ref):
    f = pl.program_id(0)
    acc = jnp.dot(xm_ref[...], w_ref[...], preferred_element_type=jnp.float32)
    post = jnp.maximum(acc + b_ref[...], 0.0)
    out_ref[...] = post
    tmax = jnp.broadcast_to(jnp.max(post, axis=1, keepdims=True), lo_ref.shape)

    @pl.when(f == 0)
    def _():
        lo_ref[...] = tmax
        hi_ref[...] = tmax

    @pl.when(f != 0)
    def _():
        lo_ref[...] = jnp.minimum(lo_ref[...], tmax)
        hi_ref[...] = jnp.maximum(hi_ref[...], tmax)


def _threshold_kernel(post_ref, lo_ref, hi_ref, tau_ref, *, k, use_lo):
    v = jax.lax.bitcast_convert_type(post_ref[...], jnp.int32)
    if use_lo:
        lo = jax.lax.bitcast_convert_type(lo_ref[:, :1], jnp.int32)
    else:
        lo = jnp.zeros((v.shape[0], 1), jnp.int32)
    hi = jax.lax.bitcast_convert_type(hi_ref[:, :1], jnp.int32) + 1

    def cond(carry):
        lo, hi = carry
        return jnp.any(hi - lo > 1)

    def body(carry):
        lo, hi = carry
        mid = lo + (hi - lo) // 2
        cnt = jnp.sum((v >= mid).astype(jnp.int32), axis=1, keepdims=True)
        ge = cnt >= k
        hi = jnp.where(cnt == k, mid + 1, jnp.where(ge, hi, mid))
        lo = jnp.where(ge, mid, lo)
        return (lo, hi)

    lo, hi = jax.lax.fori_loop(0, 8, lambda _, c: body(c), (lo, hi))
    lo, hi = jax.lax.while_loop(cond, body, (lo, hi))
    tau_ref[...] = jax.lax.bitcast_convert_type(lo, jnp.float32)


def _decode_kernel(post_ref, tau_ref, w_ref, bdec_ref, out_ref):
    f = pl.program_id(0)

    @pl.when(f == 0)
    def _():
        out_ref[...] = jnp.broadcast_to(bdec_ref[...], out_ref.shape)

    p = post_ref[...]
    masked = jnp.where(p >= tau_ref[...], p, 0.0)
    out_ref[...] += jnp.dot(masked, w_ref[...], preferred_element_type=jnp.float32)


@jax.jit
def kernel(x, W_enc, W_dec, encoder_bias, b_dec):
    ntok, act_dim = x.shape
    dict_size = W_enc.shape[1]
    k = 64

    f_tile = 512
    fd_tile = 512
    n_ftiles = dict_size // f_tile
    xm = x - b_dec[None, :]
    bias2d = encoder_bias[None, :]

    post_relu, row_lo, row_hi = pl.pallas_call(
        _encode_kernel,
        grid=(n_ftiles,),
        in_specs=[
            pl.BlockSpec((ntok, act_dim), lambda f: (0, 0)),
            pl.BlockSpec((act_dim, f_tile), lambda f: (0, f)),
            pl.BlockSpec((1, f_tile), lambda f: (0, f)),
        ],
        out_specs=[
            pl.BlockSpec((ntok, f_tile), lambda f: (0, f)),
            pl.BlockSpec((ntok, 128), lambda f: (0, 0)),
            pl.BlockSpec((ntok, 128), lambda f: (0, 0)),
        ],
        out_shape=[
            jax.ShapeDtypeStruct((ntok, dict_size), jnp.float32),
            jax.ShapeDtypeStruct((ntok, 128), jnp.float32),
            jax.ShapeDtypeStruct((ntok, 128), jnp.float32),
        ],
    )(xm, W_enc, bias2d)

    t_tile = 128
    tau = pl.pallas_call(
        functools.partial(_threshold_kernel, k=k, use_lo=n_ftiles >= k),
        grid=(ntok // t_tile,),
        in_specs=[
            pl.BlockSpec((t_tile, dict_size), lambda t: (t, 0)),
            pl.BlockSpec((t_tile, 128), lambda t: (t, 0)),
            pl.BlockSpec((t_tile, 128), lambda t: (t, 0)),
        ],
        out_specs=pl.BlockSpec((t_tile, 1), lambda t: (t, 0)),
        out_shape=jax.ShapeDtypeStruct((ntok, 1), jnp.float32),
    )(post_relu, row_lo, row_hi)

    x_hat = pl.pallas_call(
        _decode_kernel,
        grid=(dict_size // fd_tile,),
        in_specs=[
            pl.BlockSpec((ntok, fd_tile), lambda f: (0, f)),
            pl.BlockSpec((ntok, 1), lambda f: (0, 0)),
            pl.BlockSpec((fd_tile, act_dim), lambda f: (f, 0)),
            pl.BlockSpec((1, act_dim), lambda f: (0, 0)),
        ],
        out_specs=pl.BlockSpec((ntok, act_dim), lambda f: (0, 0)),
        out_shape=jax.ShapeDtypeStruct((ntok, act_dim), jnp.float32),
    )(post_relu, tau, W_dec, b_dec[None, :])

    return x_hat


# speedup vs baseline: 1.2152x; 1.0070x over previous
import functools

import jax
import jax.numpy as jnp
from jax.experimental import pallas as pl


def _encode_kernel(xm_ref, w_ref, b_ref, out_ref, lo_ref, hi_ref):
    f = pl.program_id(0)
    acc = jnp.dot(xm_ref[...], w_ref[...], preferred_element_type=jnp.float32)
    post = jnp.maximum(acc + b_ref[...], 0.0)
    out_ref[...] = post
    tmax = jnp.broadcast_to(jnp.max(post, axis=1, keepdims=True), lo_ref.shape)

    @pl.when(f == 0)
    def _():
        lo_ref[...] = tmax
        hi_ref[...] = tmax

    @pl.when(f != 0)
    def _():
        lo_ref[...] = jnp.minimum(lo_ref[...], tmax)
        hi_ref[...] = jnp.maximum(hi_ref[...], tmax)


def _threshold_kernel(post_ref, lo_ref, hi_ref, tau_ref, *, k, use_lo):
    v = jax.lax.bitcast_convert_type(post_ref[...], jnp.int32)
    if use_lo:
        lo = jax.lax.bitcast_convert_type(lo_ref[:, :1], jnp.int32)
    else:
        lo = jnp.zeros((v.shape[0], 1), jnp.int32)
    hi = jax.lax.bitcast_convert_type(hi_ref[:, :1], jnp.int32) + 1

    def cond(carry):
        lo, hi = carry
        return jnp.any(hi - lo > 1)

    def body(carry):
        lo, hi = carry
        mid = lo + (hi - lo) // 2
        cnt = jnp.sum((v >= mid).astype(jnp.int32), axis=1, keepdims=True)
        ge = cnt >= k
        hi = jnp.where(cnt == k, mid + 1, jnp.where(ge, hi, mid))
        lo = jnp.where(ge, mid, lo)
        return (lo, hi)

    lo, hi = jax.lax.fori_loop(0, 12, lambda _, c: body(c), (lo, hi))
    lo, hi = jax.lax.while_loop(cond, body, (lo, hi))
    tau_ref[...] = jax.lax.bitcast_convert_type(lo, jnp.float32)


def _decode_kernel(post_ref, tau_ref, w_ref, bdec_ref, out_ref):
    f = pl.program_id(0)

    @pl.when(f == 0)
    def _():
        out_ref[...] = jnp.broadcast_to(bdec_ref[...], out_ref.shape)

    p = post_ref[...]
    masked = jnp.where(p >= tau_ref[...], p, 0.0)
    out_ref[...] += jnp.dot(masked, w_ref[...], preferred_element_type=jnp.float32)


@jax.jit
def kernel(x, W_enc, W_dec, encoder_bias, b_dec):
    ntok, act_dim = x.shape
    dict_size = W_enc.shape[1]
    k = 64

    f_tile = 512
    fd_tile = 512
    n_ftiles = dict_size // f_tile
    xm = x - b_dec[None, :]
    bias2d = encoder_bias[None, :]

    post_relu, row_lo, row_hi = pl.pallas_call(
        _encode_kernel,
        grid=(n_ftiles,),
        in_specs=[
            pl.BlockSpec((ntok, act_dim), lambda f: (0, 0)),
            pl.BlockSpec((act_dim, f_tile), lambda f: (0, f)),
            pl.BlockSpec((1, f_tile), lambda f: (0, f)),
        ],
        out_specs=[
            pl.BlockSpec((ntok, f_tile), lambda f: (0, f)),
            pl.BlockSpec((ntok, 128), lambda f: (0, 0)),
            pl.BlockSpec((ntok, 128), lambda f: (0, 0)),
        ],
        out_shape=[
            jax.ShapeDtypeStruct((ntok, dict_size), jnp.float32),
            jax.ShapeDtypeStruct((ntok, 128), jnp.float32),
            jax.ShapeDtypeStruct((ntok, 128), jnp.float32),
        ],
    )(xm, W_enc, bias2d)

    t_tile = 128
    tau = pl.pallas_call(
        functools.partial(_threshold_kernel, k=k, use_lo=n_ftiles >= k),
        grid=(ntok // t_tile,),
        in_specs=[
            pl.BlockSpec((t_tile, dict_size), lambda t: (t, 0)),
            pl.BlockSpec((t_tile, 128), lambda t: (t, 0)),
            pl.BlockSpec((t_tile, 128), lambda t: (t, 0)),
        ],
        out_specs=pl.BlockSpec((t_tile, 1), lambda t: (t, 0)),
        out_shape=jax.ShapeDtypeStruct((ntok, 1), jnp.float32),
    )(post_relu, row_lo, row_hi)

    x_hat = pl.pallas_call(
        _decode_kernel,
        grid=(dict_size // fd_tile,),
        in_specs=[
            pl.BlockSpec((ntok, fd_tile), lambda f: (0, f)),
            pl.BlockSpec((ntok, 1), lambda f: (0, 0)),
            pl.BlockSpec((fd_tile, act_dim), lambda f: (f, 0)),
            pl.BlockSpec((1, act_dim), lambda f: (0, 0)),
        ],
        out_specs=pl.BlockSpec((ntok, act_dim), lambda f: (0, 0)),
        out_shape=jax.ShapeDtypeStruct((ntok, act_dim), jnp.float32),
    )(post_relu, tau, W_dec, b_dec[None, :])

    return x_hat
